# Initial kernel scaffold; baseline (speedup 1.0000x reference)
import jax
import jax.numpy as jnp
from jax.experimental import pallas as pl


def kernel(indices, embedding_weight):
    raise NotImplementedError("write your pallas kernel here")


# speedup vs baseline: 4.6654x; 4.6654x over previous
import functools

import jax
import jax.numpy as jnp
from jax import lax
from jax.experimental import pallas as pl
from jax.experimental.pallas import tpu as pltpu
from jax.experimental.pallas import tpu_sc as plsc

TOKEN_DIM = 64
NUM_CORES = 2
NUM_SUBCORES = 16
NUM_WORKERS = NUM_CORES * NUM_SUBCORES
TOTAL_ROWS = 4096 * 50
ROWS_PER_WORKER = TOTAL_ROWS // NUM_WORKERS
NUM_CHUNKS = 10
CHUNK = ROWS_PER_WORKER // NUM_CHUNKS

_mesh = plsc.VectorSubcoreMesh(core_axis_name="c", subcore_axis_name="s")


@functools.partial(
    pl.kernel,
    out_type=jax.ShapeDtypeStruct((TOTAL_ROWS, TOKEN_DIM), jnp.float32),
    mesh=_mesh,
    compiler_params=pltpu.CompilerParams(use_tc_tiling_on_sc=False),
    scratch_types=[
        pltpu.VMEM((ROWS_PER_WORKER,), jnp.int32),
        pltpu.VMEM((CHUNK, TOKEN_DIM), jnp.float32),
        pltpu.VMEM((CHUNK, TOKEN_DIM), jnp.float32),
        pltpu.SemaphoreType.DMA,
        pltpu.SemaphoreType.DMA,
        pltpu.SemaphoreType.DMA,
        pltpu.SemaphoreType.DMA,
    ],
)
def _sc_gather(table_hbm, idx_hbm, out_hbm, idx_v, buf0, buf1, g0, g1, o0, o1):
    wid = lax.axis_index("s") * NUM_CORES + lax.axis_index("c")
    base = wid * ROWS_PER_WORKER
    pltpu.sync_copy(idx_hbm.at[pl.ds(base, ROWS_PER_WORKER)], idx_v)

    bufs = (buf0, buf1)
    gsems = (g0, g1)
    osems = (o0, o1)

    def start_gather(c):
        b = c % 2
        return pltpu.async_copy(
            table_hbm.at[idx_v.at[pl.ds(c * CHUNK, CHUNK)]], bufs[b], gsems[b]
        )

    def start_out(c):
        b = c % 2
        return pltpu.async_copy(
            bufs[b], out_hbm.at[pl.ds(base + c * CHUNK, CHUNK)], osems[b]
        )

    g = [None, None]
    o = [None, None]
    g[0] = start_gather(0)
    for c in range(NUM_CHUNKS):
        b = c % 2
        nb = (c + 1) % 2
        if c + 1 < NUM_CHUNKS:
            if o[nb] is not None:
                o[nb].wait()
            g[nb] = start_gather(c + 1)
        g[b].wait()
        o[b] = start_out(c)
    o[0].wait()
    o[1].wait()


def kernel(indices, embedding_weight):
    flat = indices.reshape(-1).astype(jnp.int32)
    out = _sc_gather(embedding_weight, flat)
    return out.reshape(indices.shape + (TOKEN_DIM,))


# speedup vs baseline: 4.6728x; 1.0016x over previous
import functools

import jax
import jax.numpy as jnp
from jax import lax
from jax.experimental import pallas as pl
from jax.experimental.pallas import tpu as pltpu
from jax.experimental.pallas import tpu_sc as plsc

TOKEN_DIM = 64
NUM_CORES = 2
NUM_SUBCORES = 16
NUM_WORKERS = NUM_CORES * NUM_SUBCORES
TOTAL_ROWS = 4096 * 50
ROWS_PER_WORKER = TOTAL_ROWS // NUM_WORKERS
NUM_CHUNKS = 16
CHUNK = ROWS_PER_WORKER // NUM_CHUNKS
NBUF = 4
AHEAD = 2

_mesh = plsc.VectorSubcoreMesh(core_axis_name="c", subcore_axis_name="s")


@functools.partial(
    pl.kernel,
    out_type=jax.ShapeDtypeStruct((TOTAL_ROWS, TOKEN_DIM), jnp.float32),
    mesh=_mesh,
    compiler_params=pltpu.CompilerParams(use_tc_tiling_on_sc=False),
    scratch_types=[
        pltpu.VMEM((ROWS_PER_WORKER,), jnp.int32),
        *[pltpu.VMEM((CHUNK, TOKEN_DIM), jnp.float32) for _ in range(NBUF)],
        *[pltpu.SemaphoreType.DMA for _ in range(2 * NBUF)],
    ],
)
def _sc_gather(table_hbm, idx_hbm, out_hbm, idx_v, *rest):
    bufs = rest[:NBUF]
    gsems = rest[NBUF : 2 * NBUF]
    osems = rest[2 * NBUF :]

    wid = lax.axis_index("s") * NUM_CORES + lax.axis_index("c")
    base = wid * ROWS_PER_WORKER
    pltpu.sync_copy(idx_hbm.at[pl.ds(base, ROWS_PER_WORKER)], idx_v)

    def start_gather(c):
        b = c % NBUF
        return pltpu.async_copy(
            table_hbm.at[idx_v.at[pl.ds(c * CHUNK, CHUNK)]], bufs[b], gsems[b]
        )

    def start_out(c):
        b = c % NBUF
        return pltpu.async_copy(
            bufs[b], out_hbm.at[pl.ds(base + c * CHUNK, CHUNK)], osems[b]
        )

    g = [None] * NBUF
    o = [None] * NBUF
    for c in range(AHEAD):
        g[c] = start_gather(c)
    for c in range(NUM_CHUNKS):
        nc = c + AHEAD
        if nc < NUM_CHUNKS:
            b2 = nc % NBUF
            if o[b2] is not None:
                o[b2].wait()
            g[b2] = start_gather(nc)
        b = c % NBUF
        g[b].wait()
        o[b] = start_out(c)
    for b in range(NBUF):
        if o[b] is not None:
            o[b].wait()


def kernel(indices, embedding_weight):
    flat = indices.reshape(-1).astype(jnp.int32)
    out = _sc_gather(embedding_weight, flat)
    return out.reshape(indices.shape + (TOKEN_DIM,))
